# baseline (speedup 1.0000x reference)
import functools

import jax
import jax.numpy as jnp
from jax import lax
from jax.experimental import pallas as pl
from jax.experimental.pallas import tpu as pltpu
from jax.experimental.pallas import tpu_sc as plsc

NUM_PIDS = 1024
SEG = 8192
LEN = 128
TOTAL = SEG * LEN

R_GAS = 0.0083144598
KELVIN = 273.15
TROOM = 25.0 + KELVIN
DHA_RD = 46.39
DHD_VCMAX = 200.0
DHD_JMAX = 200.0
DHD_TPU = 201.8

NC = 2
NS = 16
LANES = 16
NW = NC * NS
SEG_PER_W = SEG // NW


def _sc_gather_body(pids_hbm, dV_hbm, dJ_hbm, dT_hbm, tV_hbm, tJ_hbm, tT_hbm,
                    o_a1V, o_a2V, o_tpV, o_a1J, o_a2J, o_tpJ,
                    o_a1T, o_a2T, o_tpT,
                    pids_v, pids8_v, dVv, dJv, dTv, tVv, tJv, tTv,
                    v_a1V, v_a2V, v_tpV, v_a1J, v_a2J, v_tpJ,
                    v_a1T, v_a2T, v_tpT, sem):
    wid = lax.axis_index("s") * NC + lax.axis_index("c")
    base = wid * SEG_PER_W
    descs = [
        pltpu.async_copy(pids_hbm.at[pl.ds(base, SEG_PER_W)], pids_v, sem),
        pltpu.async_copy(pids_hbm.at[pl.ds(0, LANES)], pids8_v, sem),
        pltpu.async_copy(dV_hbm, dVv, sem),
        pltpu.async_copy(dJ_hbm, dJv, sem),
        pltpu.async_copy(dT_hbm, dTv, sem),
        pltpu.async_copy(tV_hbm, tVv, sem),
        pltpu.async_copy(tJ_hbm, tJv, sem),
        pltpu.async_copy(tT_hbm, tTv, sem),
    ]
    for d in descs:
        d.wait()
    bufs = (v_a1V, v_a2V, v_tpV, v_a1J, v_a2J, v_tpJ, v_a1T, v_a2T, v_tpT)
    for i in range(SEG_PER_W // LANES):
        sl = pl.ds(i * LANES, LANES)
        p = pids_v[sl]
        q = plsc.load_gather(pids8_v, [jnp.right_shift(p, 7)])
        for ch, (dv, tv) in enumerate(((dVv, tVv), (dJv, tJv), (dTv, tTv))):
            bufs[3 * ch + 0][sl] = plsc.load_gather(dv, [p])
            bufs[3 * ch + 1][sl] = plsc.load_gather(dv, [q])
            bufs[3 * ch + 2][sl] = plsc.load_gather(tv, [p])
    o_refs = (o_a1V, o_a2V, o_tpV, o_a1J, o_a2J, o_tpJ, o_a1T, o_a2T, o_tpT)
    outs = [
        pltpu.async_copy(v, o.at[pl.ds(base, SEG_PER_W)], sem)
        for v, o in zip(bufs, o_refs)
    ]
    for d in outs:
        d.wait()


def _sc_gather(pids, dV, dJ, dT, tV, tJ, tT):
    mesh = plsc.VectorSubcoreMesh(core_axis_name="c", subcore_axis_name="s",
                                  num_cores=NC, num_subcores=NS)
    return pl.kernel(
        _sc_gather_body,
        out_type=tuple(jax.ShapeDtypeStruct((SEG,), jnp.float32)
                       for _ in range(9)),
        mesh=mesh,
        compiler_params=pltpu.CompilerParams(needs_layout_passes=False),
        scratch_types=[
            pltpu.VMEM((SEG_PER_W,), jnp.int32),
            pltpu.VMEM((LANES,), jnp.int32),
        ] + [pltpu.VMEM((NUM_PIDS,), jnp.float32) for _ in range(6)]
          + [pltpu.VMEM((SEG_PER_W,), jnp.float32) for _ in range(9)]
          + [pltpu.SemaphoreType.DMA],
    )(pids, dV, dJ, dT, tV, tJ, tT)


BS = 128


def _col(row):
    ones = jnp.ones((1, 1), dtype=jnp.float32)
    return lax.dot_general(row, ones, (((0,), (0,)), ((), ())),
                           preferred_element_type=jnp.float32)


def _tc_body(tleaf, vc25, jm25, tp25, rd25,
             a1V, a2V, tpV, a1J, a2J, tpJ, a1T, a2T, tpT, out_ref):
    coefs = (a1V, a2V, tpV, a1J, a2J, tpJ, a1T, a2T, tpT)
    c_rk = jnp.float32(1.0 / (R_GAS * TROOM))
    c_r = jnp.float32(1.0 / R_GAS)
    rec_troom = jnp.float32(1.0 / TROOM)
    d_vj = jnp.float32(DHD_VCMAX / R_GAS)
    d_t = jnp.float32(DHD_TPU / R_GAS)

    r = 1.0 / tleaf[...]
    e_vj = jnp.exp(-d_vj * r)
    e_t = jnp.exp(-d_t * r)

    def chan(k25, ch, dhd, dhd_r, e):
        a1 = _col(coefs[3 * ch + 0][0])
        a2 = _col(coefs[3 * ch + 1][0])
        tp = _col(coefs[3 * ch + 2][0])
        g = a1 / (jnp.float32(dhd) - a1)
        A = a2 * c_rk
        B = a2 * c_r
        rtp = 1.0 / tp
        numc = 1.0 + g * jnp.exp(dhd_r * (rtp - rec_troom))
        G = g * jnp.exp(dhd_r * rtp)
        return k25[...] * numc * jnp.exp(A - B * r) / (1.0 + G * e)

    out_ref[0] = chan(vc25, 0, DHD_VCMAX, d_vj, e_vj)
    out_ref[1] = chan(jm25, 1, DHD_JMAX, d_vj, e_vj)
    out_ref[2] = chan(tp25, 2, DHD_TPU, d_t, e_t)
    ard = jnp.float32(DHA_RD / (R_GAS * TROOM))
    brd = jnp.float32(DHA_RD / R_GAS)
    out_ref[3] = rd25[...] * jnp.exp(ard - brd * r)


def kernel(Tleaf, Vcmax25, Jmax25, TPU25, Rd25, dHa_Vcmax, dHa_Jmax, dHa_TPU,
           Topt_Vcmax, Topt_Jmax, Topt_TPU, PIDs, lengths):
    del lengths
    coefs = _sc_gather(PIDs, dHa_Vcmax, dHa_Jmax, dHa_TPU,
                       Topt_Vcmax, Topt_Jmax, Topt_TPU)
    coefs = [c.reshape(SEG // LEN, 1, LEN) for c in coefs]
    elems = [x.reshape(SEG, LEN) for x in (Tleaf, Vcmax25, Jmax25, TPU25, Rd25)]

    eblk = pl.BlockSpec((BS, LEN), lambda i: (i, 0))
    cblk = pl.BlockSpec((1, 1, LEN), lambda i: (i, 0, 0))
    out = pl.pallas_call(
        _tc_body,
        grid=(SEG // BS,),
        in_specs=[eblk] * 5 + [cblk] * 9,
        out_specs=pl.BlockSpec((4, BS, LEN), lambda i: (0, i, 0)),
        out_shape=jax.ShapeDtypeStruct((4, SEG, LEN), jnp.float32),
    )(*elems, *coefs)
    return out.reshape(4, TOTAL)


# speedup vs baseline: 1.0608x; 1.0608x over previous
import functools

import jax
import jax.numpy as jnp
from jax import lax
from jax.experimental import pallas as pl
from jax.experimental.pallas import tpu as pltpu
from jax.experimental.pallas import tpu_sc as plsc

NUM_PIDS = 1024
SEG = 8192
LEN = 128
TOTAL = SEG * LEN

R_GAS = 0.0083144598
KELVIN = 273.15
TROOM = 25.0 + KELVIN
DHA_RD = 46.39
DHD_VCMAX = 200.0
DHD_JMAX = 200.0
DHD_TPU = 201.8

NC = 2
NS = 16
LANES = 16
NW = NC * NS
SEG_PER_W = SEG // NW


def _sc_gather_body(pids_hbm, dV_hbm, dJ_hbm, dT_hbm, tV_hbm, tJ_hbm, tT_hbm,
                    o_a1V, o_a2V, o_tpV, o_a1J, o_a2J, o_tpJ,
                    o_a1T, o_a2T, o_tpT,
                    pids_v, pids8_v, dVv, dJv, dTv, tVv, tJv, tTv,
                    v_a1V, v_a2V, v_tpV, v_a1J, v_a2J, v_tpJ,
                    v_a1T, v_a2T, v_tpT, sem):
    wid = lax.axis_index("s") * NC + lax.axis_index("c")
    base = wid * SEG_PER_W
    pltpu.sync_copy(pids_hbm.at[pl.ds(base, SEG_PER_W)], pids_v)
    pltpu.sync_copy(pids_hbm.at[pl.ds(0, LANES)], pids8_v)
    pltpu.sync_copy(dV_hbm, dVv)
    pltpu.sync_copy(dJ_hbm, dJv)
    pltpu.sync_copy(dT_hbm, dTv)
    pltpu.sync_copy(tV_hbm, tVv)
    pltpu.sync_copy(tJ_hbm, tJv)
    pltpu.sync_copy(tT_hbm, tTv)
    bufs = (v_a1V, v_a2V, v_tpV, v_a1J, v_a2J, v_tpJ, v_a1T, v_a2T, v_tpT)
    for i in range(SEG_PER_W // LANES):
        sl = pl.ds(i * LANES, LANES)
        p = pids_v[sl]
        q = plsc.load_gather(pids8_v, [jnp.right_shift(p, 7)])
        for ch, (dv, tv) in enumerate(((dVv, tVv), (dJv, tJv), (dTv, tTv))):
            bufs[3 * ch + 0][sl] = plsc.load_gather(dv, [p])
            bufs[3 * ch + 1][sl] = plsc.load_gather(dv, [q])
            bufs[3 * ch + 2][sl] = plsc.load_gather(tv, [p])
    o_refs = (o_a1V, o_a2V, o_tpV, o_a1J, o_a2J, o_tpJ, o_a1T, o_a2T, o_tpT)
    for v, o in zip(bufs, o_refs):
        pltpu.sync_copy(v, o.at[pl.ds(base, SEG_PER_W)])


def _sc_gather(pids, dV, dJ, dT, tV, tJ, tT):
    mesh = plsc.VectorSubcoreMesh(core_axis_name="c", subcore_axis_name="s",
                                  num_cores=NC, num_subcores=NS)
    return pl.kernel(
        _sc_gather_body,
        out_type=tuple(jax.ShapeDtypeStruct((SEG,), jnp.float32)
                       for _ in range(9)),
        mesh=mesh,
        compiler_params=pltpu.CompilerParams(needs_layout_passes=False),
        scratch_types=[
            pltpu.VMEM((SEG_PER_W,), jnp.int32),
            pltpu.VMEM((LANES,), jnp.int32),
        ] + [pltpu.VMEM((NUM_PIDS,), jnp.float32) for _ in range(6)]
          + [pltpu.VMEM((SEG_PER_W,), jnp.float32) for _ in range(9)]
          + [pltpu.SemaphoreType.DMA],
    )(pids, dV, dJ, dT, tV, tJ, tT)


BS = 128


def _col(row):
    m = lax.broadcast_in_dim(row, (BS, BS), (0, 1))
    i = lax.broadcasted_iota(jnp.int32, (BS, BS), 0)
    j = lax.broadcasted_iota(jnp.int32, (BS, BS), 1)
    d = jnp.where(i == j, m, jnp.float32(0.0))
    return jnp.sum(d, axis=1, keepdims=True)


def _tc_body(tleaf, vc25, jm25, tp25, rd25,
             a1V, a2V, tpV, a1J, a2J, tpJ, a1T, a2T, tpT, out_ref):
    coefs = (a1V, a2V, tpV, a1J, a2J, tpJ, a1T, a2T, tpT)
    c_rk = jnp.float32(1.0 / (R_GAS * TROOM))
    c_r = jnp.float32(1.0 / R_GAS)
    rec_troom = jnp.float32(1.0 / TROOM)
    d_vj = jnp.float32(DHD_VCMAX / R_GAS)
    d_t = jnp.float32(DHD_TPU / R_GAS)

    r = 1.0 / tleaf[...]
    e_vj = jnp.exp(-d_vj * r)
    e_t = jnp.exp(-d_t * r)

    def chan(k25, ch, dhd, dhd_r, e):
        a1 = _col(coefs[3 * ch + 0][0])
        a2 = _col(coefs[3 * ch + 1][0])
        tp = _col(coefs[3 * ch + 2][0])
        g = a1 / (jnp.float32(dhd) - a1)
        A = a2 * c_rk
        B = a2 * c_r
        rtp = 1.0 / tp
        numc = 1.0 + g * jnp.exp(dhd_r * (rtp - rec_troom))
        G = g * jnp.exp(dhd_r * rtp)
        return k25[...] * numc * jnp.exp(A - B * r) / (1.0 + G * e)

    out_ref[0] = chan(vc25, 0, DHD_VCMAX, d_vj, e_vj)
    out_ref[1] = chan(jm25, 1, DHD_JMAX, d_vj, e_vj)
    out_ref[2] = chan(tp25, 2, DHD_TPU, d_t, e_t)
    ard = jnp.float32(DHA_RD / (R_GAS * TROOM))
    brd = jnp.float32(DHA_RD / R_GAS)
    out_ref[3] = rd25[...] * jnp.exp(ard - brd * r)


def kernel(Tleaf, Vcmax25, Jmax25, TPU25, Rd25, dHa_Vcmax, dHa_Jmax, dHa_TPU,
           Topt_Vcmax, Topt_Jmax, Topt_TPU, PIDs, lengths):
    del lengths
    coefs = _sc_gather(PIDs, dHa_Vcmax, dHa_Jmax, dHa_TPU,
                       Topt_Vcmax, Topt_Jmax, Topt_TPU)
    coefs = [c.reshape(SEG // LEN, 1, LEN) for c in coefs]
    elems = [x.reshape(SEG, LEN) for x in (Tleaf, Vcmax25, Jmax25, TPU25, Rd25)]

    eblk = pl.BlockSpec((BS, LEN), lambda i: (i, 0))
    cblk = pl.BlockSpec((1, 1, LEN), lambda i: (i, 0, 0))
    out = pl.pallas_call(
        _tc_body,
        grid=(SEG // BS,),
        in_specs=[eblk] * 5 + [cblk] * 9,
        out_specs=pl.BlockSpec((4, BS, LEN), lambda i: (0, i, 0)),
        out_shape=jax.ShapeDtypeStruct((4, SEG, LEN), jnp.float32),
    )(*elems, *coefs)
    return out.reshape(4, TOTAL)


# speedup vs baseline: 1.2776x; 1.2044x over previous
import functools

import jax
import jax.numpy as jnp
from jax import lax
from jax.experimental import pallas as pl
from jax.experimental.pallas import tpu as pltpu
from jax.experimental.pallas import tpu_sc as plsc

NUM_PIDS = 1024
SEG = 8192
LEN = 128
TOTAL = SEG * LEN

R_GAS = 0.0083144598
KELVIN = 273.15
TROOM = 25.0 + KELVIN
DHA_RD = 46.39
DHD_VCMAX = 200.0
DHD_JMAX = 200.0
DHD_TPU = 201.8

NC = 2
NS = 16
LANES = 16
NW = NC * NS
SEG_PER_W = SEG // NW


def _sc_gather_body(pids_hbm, dV_hbm, dJ_hbm, dT_hbm, tV_hbm, tJ_hbm, tT_hbm,
                    o_a1V, o_a2V, o_tpV, o_a1J, o_a2J, o_tpJ,
                    o_a1T, o_a2T, o_tpT,
                    pids_v, pids8_v, dVv, dJv, dTv, tVv, tJv, tTv,
                    v_a1V, v_a2V, v_tpV, v_a1J, v_a2J, v_tpJ,
                    v_a1T, v_a2T, v_tpT, sem):
    wid = lax.axis_index("s") * NC + lax.axis_index("c")
    base = wid * SEG_PER_W
    pltpu.sync_copy(pids_hbm.at[pl.ds(base, SEG_PER_W)], pids_v)
    pltpu.sync_copy(pids_hbm.at[pl.ds(0, LANES)], pids8_v)
    pltpu.sync_copy(dV_hbm, dVv)
    pltpu.sync_copy(dJ_hbm, dJv)
    pltpu.sync_copy(dT_hbm, dTv)
    pltpu.sync_copy(tV_hbm, tVv)
    pltpu.sync_copy(tJ_hbm, tJv)
    pltpu.sync_copy(tT_hbm, tTv)
    bufs = (v_a1V, v_a2V, v_tpV, v_a1J, v_a2J, v_tpJ, v_a1T, v_a2T, v_tpT)
    for i in range(SEG_PER_W // LANES):
        sl = pl.ds(i * LANES, LANES)
        p = pids_v[sl]
        q = plsc.load_gather(pids8_v, [jnp.right_shift(p, 7)])
        for ch, (dv, tv) in enumerate(((dVv, tVv), (dJv, tJv), (dTv, tTv))):
            bufs[3 * ch + 0][sl] = plsc.load_gather(dv, [p])
            bufs[3 * ch + 1][sl] = plsc.load_gather(dv, [q])
            bufs[3 * ch + 2][sl] = plsc.load_gather(tv, [p])
    o_refs = (o_a1V, o_a2V, o_tpV, o_a1J, o_a2J, o_tpJ, o_a1T, o_a2T, o_tpT)
    for v, o in zip(bufs, o_refs):
        pltpu.sync_copy(v, o.at[pl.ds(base, SEG_PER_W)])


def _sc_gather(pids, dV, dJ, dT, tV, tJ, tT):
    mesh = plsc.VectorSubcoreMesh(core_axis_name="c", subcore_axis_name="s",
                                  num_cores=NC, num_subcores=NS)
    return pl.kernel(
        _sc_gather_body,
        out_type=tuple(jax.ShapeDtypeStruct((SEG,), jnp.float32)
                       for _ in range(9)),
        mesh=mesh,
        compiler_params=pltpu.CompilerParams(needs_layout_passes=False),
        scratch_types=[
            pltpu.VMEM((SEG_PER_W,), jnp.int32),
            pltpu.VMEM((LANES,), jnp.int32),
        ] + [pltpu.VMEM((NUM_PIDS,), jnp.float32) for _ in range(6)]
          + [pltpu.VMEM((SEG_PER_W,), jnp.float32) for _ in range(9)]
          + [pltpu.SemaphoreType.DMA],
    )(pids, dV, dJ, dT, tV, tJ, tT)


BS = 128


def _col(row):
    m = lax.broadcast_in_dim(row, (BS, BS), (0, 1))
    i = lax.broadcasted_iota(jnp.int32, (BS, BS), 0)
    j = lax.broadcasted_iota(jnp.int32, (BS, BS), 1)
    d = jnp.where(i == j, m, jnp.float32(0.0))
    return jnp.sum(d, axis=1, keepdims=True)


def _tc_body(tleaf, vc25, jm25, tp25, rd25,
             a1V, a2V, tpV, a1J, a2J, tpJ, a1T, a2T, tpT, out_ref):
    coefs = (a1V, a2V, tpV, a1J, a2J, tpJ, a1T, a2T, tpT)
    chunk = BS * LEN
    c_rk = jnp.float32(1.0 / (R_GAS * TROOM))
    c_r = jnp.float32(1.0 / R_GAS)
    rec_troom = jnp.float32(1.0 / TROOM)
    d_vj = jnp.float32(DHD_VCMAX / R_GAS)
    d_t = jnp.float32(DHD_TPU / R_GAS)

    r = 1.0 / tleaf[...].reshape(BS, LEN)
    e_vj = jnp.exp(-d_vj * r)
    e_t = jnp.exp(-d_t * r)

    def chan(k25, ch, dhd, dhd_r, e):
        a1 = _col(coefs[3 * ch + 0][...].reshape(1, BS))
        a2 = _col(coefs[3 * ch + 1][...].reshape(1, BS))
        tp = _col(coefs[3 * ch + 2][...].reshape(1, BS))
        g = a1 / (jnp.float32(dhd) - a1)
        A = a2 * c_rk
        B = a2 * c_r
        rtp = 1.0 / tp
        numc = 1.0 + g * jnp.exp(dhd_r * (rtp - rec_troom))
        G = g * jnp.exp(dhd_r * rtp)
        k25 = k25[...].reshape(BS, LEN)
        return k25 * numc * jnp.exp(A - B * r) / (1.0 + G * e)

    out_ref[0] = chan(vc25, 0, DHD_VCMAX, d_vj, e_vj).reshape(chunk)
    out_ref[1] = chan(jm25, 1, DHD_JMAX, d_vj, e_vj).reshape(chunk)
    out_ref[2] = chan(tp25, 2, DHD_TPU, d_t, e_t).reshape(chunk)
    ard = jnp.float32(DHA_RD / (R_GAS * TROOM))
    brd = jnp.float32(DHA_RD / R_GAS)
    out_ref[3] = (rd25[...].reshape(BS, LEN)
                  * jnp.exp(ard - brd * r)).reshape(chunk)


def kernel(Tleaf, Vcmax25, Jmax25, TPU25, Rd25, dHa_Vcmax, dHa_Jmax, dHa_TPU,
           Topt_Vcmax, Topt_Jmax, Topt_TPU, PIDs, lengths):
    del lengths
    coefs = _sc_gather(PIDs, dHa_Vcmax, dHa_Jmax, dHa_TPU,
                       Topt_Vcmax, Topt_Jmax, Topt_TPU)
    elems = (Tleaf, Vcmax25, Jmax25, TPU25, Rd25)

    chunk = BS * LEN
    eblk = pl.BlockSpec((chunk,), lambda i: (i,))
    cblk = pl.BlockSpec((BS,), lambda i: (i,))
    return pl.pallas_call(
        _tc_body,
        grid=(SEG // BS,),
        in_specs=[eblk] * 5 + [cblk] * 9,
        out_specs=pl.BlockSpec((4, chunk), lambda i: (0, i)),
        out_shape=jax.ShapeDtypeStruct((4, TOTAL), jnp.float32),
    )(*elems, *coefs)


# speedup vs baseline: 1.3352x; 1.0451x over previous
import functools

import jax
import jax.numpy as jnp
from jax import lax
from jax.experimental import pallas as pl
from jax.experimental.pallas import tpu as pltpu
from jax.experimental.pallas import tpu_sc as plsc

NUM_PIDS = 1024
SEG = 8192
LEN = 128
TOTAL = SEG * LEN

R_GAS = 0.0083144598
KELVIN = 273.15
TROOM = 25.0 + KELVIN
DHA_RD = 46.39
DHD_VCMAX = 200.0
DHD_JMAX = 200.0
DHD_TPU = 201.8

NC = 2
NS = 16
LANES = 16
NW = NC * NS
SEG_PER_W = SEG // NW


NCOEF = 9
NCOEF_PAD = 16
BLK_PER_W = SEG_PER_W // LEN
CHUNK_C = NCOEF_PAD * LEN


def _sc_gather_body(pids_hbm, dV_hbm, dJ_hbm, dT_hbm, tV_hbm, tJ_hbm, tT_hbm,
                    coef_hbm,
                    pids_v, pids8_v, dVv, dJv, dTv, tVv, tJv, tTv,
                    buf, sem):
    wid = lax.axis_index("s") * NC + lax.axis_index("c")
    base = wid * SEG_PER_W
    descs = [
        pltpu.async_copy(pids_hbm.at[pl.ds(base, SEG_PER_W)], pids_v, sem),
        pltpu.async_copy(pids_hbm.at[pl.ds(0, LANES)], pids8_v, sem),
        pltpu.async_copy(dV_hbm, dVv, sem),
        pltpu.async_copy(dJ_hbm, dJv, sem),
        pltpu.async_copy(dT_hbm, dTv, sem),
        pltpu.async_copy(tV_hbm, tVv, sem),
        pltpu.async_copy(tJ_hbm, tJv, sem),
        pltpu.async_copy(tT_hbm, tTv, sem),
    ]
    for d in descs:
        d.wait()
    for i in range(SEG_PER_W // LANES):
        p = pids_v[pl.ds(i * LANES, LANES)]
        q = plsc.load_gather(pids8_v, [jnp.right_shift(p, 7)])
        off = (i // 8) * CHUNK_C + (i % 8) * LANES
        for ch, (dv, tv) in enumerate(((dVv, tVv), (dJv, tJv), (dTv, tTv))):
            buf[pl.ds(off + (3 * ch + 0) * LEN, LANES)] = \
                plsc.load_gather(dv, [p])
            buf[pl.ds(off + (3 * ch + 1) * LEN, LANES)] = \
                plsc.load_gather(dv, [q])
            buf[pl.ds(off + (3 * ch + 2) * LEN, LANES)] = \
                plsc.load_gather(tv, [p])
    pltpu.sync_copy(
        buf, coef_hbm.at[pl.ds(wid * BLK_PER_W * CHUNK_C,
                               BLK_PER_W * CHUNK_C)])


def _sc_gather(pids, dV, dJ, dT, tV, tJ, tT):
    mesh = plsc.VectorSubcoreMesh(core_axis_name="c", subcore_axis_name="s",
                                  num_cores=NC, num_subcores=NS)
    return pl.kernel(
        _sc_gather_body,
        out_type=jax.ShapeDtypeStruct((SEG // LEN * CHUNK_C,), jnp.float32),
        mesh=mesh,
        compiler_params=pltpu.CompilerParams(needs_layout_passes=False),
        scratch_types=[
            pltpu.VMEM((SEG_PER_W,), jnp.int32),
            pltpu.VMEM((LANES,), jnp.int32),
        ] + [pltpu.VMEM((NUM_PIDS,), jnp.float32) for _ in range(6)]
          + [pltpu.VMEM((BLK_PER_W * CHUNK_C,), jnp.float32),
             pltpu.SemaphoreType.DMA],
    )(pids, dV, dJ, dT, tV, tJ, tT)


BS = 128


def _col(row):
    m = lax.broadcast_in_dim(row, (BS, BS), (0, 1))
    i = lax.broadcasted_iota(jnp.int32, (BS, BS), 0)
    j = lax.broadcasted_iota(jnp.int32, (BS, BS), 1)
    d = jnp.where(i == j, m, jnp.float32(0.0))
    return jnp.sum(d, axis=1, keepdims=True)


def _tc_body(tleaf, vc25, jm25, tp25, rd25, coef, out_ref):
    chunk = BS * LEN
    crows = coef[...].reshape(NCOEF_PAD, LEN)
    c_rk = jnp.float32(1.0 / (R_GAS * TROOM))
    c_r = jnp.float32(1.0 / R_GAS)
    rec_troom = jnp.float32(1.0 / TROOM)
    d_vj = jnp.float32(DHD_VCMAX / R_GAS)
    d_t = jnp.float32(DHD_TPU / R_GAS)

    r = 1.0 / tleaf[...].reshape(BS, LEN)
    e_vj = jnp.exp(-d_vj * r)
    e_t = jnp.exp(-d_t * r)

    def chan(k25, ch, dhd, dhd_r, e):
        a1 = _col(crows[3 * ch + 0:3 * ch + 1, :])
        a2 = _col(crows[3 * ch + 1:3 * ch + 2, :])
        tp = _col(crows[3 * ch + 2:3 * ch + 3, :])
        g = a1 / (jnp.float32(dhd) - a1)
        A = a2 * c_rk
        B = a2 * c_r
        rtp = 1.0 / tp
        numc = 1.0 + g * jnp.exp(dhd_r * (rtp - rec_troom))
        G = g * jnp.exp(dhd_r * rtp)
        k25 = k25[...].reshape(BS, LEN)
        return k25 * numc * jnp.exp(A - B * r) / (1.0 + G * e)

    out_ref[0] = chan(vc25, 0, DHD_VCMAX, d_vj, e_vj).reshape(chunk)
    out_ref[1] = chan(jm25, 1, DHD_JMAX, d_vj, e_vj).reshape(chunk)
    out_ref[2] = chan(tp25, 2, DHD_TPU, d_t, e_t).reshape(chunk)
    ard = jnp.float32(DHA_RD / (R_GAS * TROOM))
    brd = jnp.float32(DHA_RD / R_GAS)
    out_ref[3] = (rd25[...].reshape(BS, LEN)
                  * jnp.exp(ard - brd * r)).reshape(chunk)


def kernel(Tleaf, Vcmax25, Jmax25, TPU25, Rd25, dHa_Vcmax, dHa_Jmax, dHa_TPU,
           Topt_Vcmax, Topt_Jmax, Topt_TPU, PIDs, lengths):
    del lengths
    coefs = _sc_gather(PIDs, dHa_Vcmax, dHa_Jmax, dHa_TPU,
                       Topt_Vcmax, Topt_Jmax, Topt_TPU)
    elems = (Tleaf, Vcmax25, Jmax25, TPU25, Rd25)

    chunk = BS * LEN
    eblk = pl.BlockSpec((chunk,), lambda i: (i,))
    cblk = pl.BlockSpec((BS // LEN * CHUNK_C,), lambda i: (i,))
    return pl.pallas_call(
        _tc_body,
        grid=(SEG // BS,),
        in_specs=[eblk] * 5 + [cblk],
        out_specs=pl.BlockSpec((4, chunk), lambda i: (0, i)),
        out_shape=jax.ShapeDtypeStruct((4, TOTAL), jnp.float32),
    )(*elems, coefs)


# speedup vs baseline: 1.9514x; 1.4615x over previous
import functools

import jax
import jax.numpy as jnp
from jax import lax
from jax.experimental import pallas as pl
from jax.experimental.pallas import tpu as pltpu
from jax.experimental.pallas import tpu_sc as plsc

NUM_PIDS = 1024
SEG = 8192
LEN = 128
TOTAL = SEG * LEN

R_GAS = 0.0083144598
KELVIN = 273.15
TROOM = 25.0 + KELVIN
DHA_RD = 46.39
DHD_VCMAX = 200.0
DHD_JMAX = 200.0
DHD_TPU = 201.8

NC = 2
NS = 16
LANES = 16
NW = NC * NS
SEG_PER_W = SEG // NW


NCOEF = 9
NCOEF_PAD = 16
BLK_PER_W = SEG_PER_W // LEN
CHUNK_C = NCOEF_PAD * LEN


def _sc_gather_body(pids_hbm, dV_hbm, dJ_hbm, dT_hbm, tV_hbm, tJ_hbm, tT_hbm,
                    coef_hbm,
                    pids_v, pids8_v, dVv, dJv, dTv, tVv, tJv, tTv,
                    buf, sem):
    wid = lax.axis_index("s") * NC + lax.axis_index("c")
    base = wid * SEG_PER_W
    descs = [
        pltpu.async_copy(pids_hbm.at[pl.ds(base, SEG_PER_W)], pids_v, sem),
        pltpu.async_copy(pids_hbm.at[pl.ds(0, LANES)], pids8_v, sem),
        pltpu.async_copy(dV_hbm, dVv, sem),
        pltpu.async_copy(dJ_hbm, dJv, sem),
        pltpu.async_copy(dT_hbm, dTv, sem),
        pltpu.async_copy(tV_hbm, tVv, sem),
        pltpu.async_copy(tJ_hbm, tJv, sem),
        pltpu.async_copy(tT_hbm, tTv, sem),
    ]
    for d in descs:
        d.wait()
    for i in range(SEG_PER_W // LANES):
        p = pids_v[pl.ds(i * LANES, LANES)]
        q = plsc.load_gather(pids8_v, [jnp.right_shift(p, 7)])
        off = (i // 8) * CHUNK_C + (i % 8) * LANES
        for ch, (dv, tv) in enumerate(((dVv, tVv), (dJv, tJv), (dTv, tTv))):
            buf[pl.ds(off + (3 * ch + 0) * LEN, LANES)] = \
                plsc.load_gather(dv, [p])
            buf[pl.ds(off + (3 * ch + 1) * LEN, LANES)] = \
                plsc.load_gather(dv, [q])
            buf[pl.ds(off + (3 * ch + 2) * LEN, LANES)] = \
                plsc.load_gather(tv, [p])
    pltpu.sync_copy(
        buf, coef_hbm.at[pl.ds(wid * BLK_PER_W * CHUNK_C,
                               BLK_PER_W * CHUNK_C)])


def _sc_gather(pids, dV, dJ, dT, tV, tJ, tT):
    mesh = plsc.VectorSubcoreMesh(core_axis_name="c", subcore_axis_name="s",
                                  num_cores=NC, num_subcores=NS)
    return pl.kernel(
        _sc_gather_body,
        out_type=jax.ShapeDtypeStruct((SEG // LEN * CHUNK_C,), jnp.float32),
        mesh=mesh,
        compiler_params=pltpu.CompilerParams(needs_layout_passes=False),
        scratch_types=[
            pltpu.VMEM((SEG_PER_W,), jnp.int32),
            pltpu.VMEM((LANES,), jnp.int32),
        ] + [pltpu.VMEM((NUM_PIDS,), jnp.float32) for _ in range(6)]
          + [pltpu.VMEM((BLK_PER_W * CHUNK_C,), jnp.float32),
             pltpu.SemaphoreType.DMA],
    )(pids, dV, dJ, dT, tV, tJ, tT)


NB = 4
BS = NB * LEN
LOG2E = 1.4426950408889634


def _tc_body(tleaf, vc25, jm25, tp25, rd25, coef, out_ref):
    chunk = BS * LEN
    sub = LEN * LEN
    c_rk = jnp.float32(1.0 / (R_GAS * TROOM))
    c_r = jnp.float32(1.0 / R_GAS)
    rec_troom = jnp.float32(1.0 / TROOM)
    d_vj = jnp.float32(DHD_VCMAX / R_GAS)
    d_t = jnp.float32(DHD_TPU / R_GAS)
    chans = ((DHD_VCMAX, d_vj), (DHD_JMAX, d_vj), (DHD_TPU, d_t))

    def exp2(x):
        return jnp.exp2(x)

    coef2d = coef[...].reshape(NB * NCOEF_PAD, LEN)
    rows = []
    for b in range(NB):
        cb = b * NCOEF_PAD
        for ch, (dhd, dhd_r) in enumerate(chans):
            a1 = coef2d[cb + 3 * ch + 0:cb + 3 * ch + 1, :]
            a2 = coef2d[cb + 3 * ch + 1:cb + 3 * ch + 2, :]
            tp = coef2d[cb + 3 * ch + 2:cb + 3 * ch + 3, :]
            g = a1 / (jnp.float32(dhd) - a1)
            rtp = 1.0 / tp
            rows.append(a2 * c_rk)
            rows.append(a2 * (c_r * LOG2E))
            rows.append(1.0 + g * exp2((dhd_r * LOG2E) * (rtp - rec_troom)))
            rows.append(g * exp2((dhd_r * LOG2E) * rtp))
    D = jnp.concatenate(rows, axis=0)
    ii = lax.broadcasted_iota(jnp.int32, (LEN, LEN), 0)
    jj = lax.broadcasted_iota(jnp.int32, (LEN, LEN), 1)
    ident = jnp.where(ii == jj, jnp.float32(1.0), jnp.float32(0.0))
    T = lax.dot_general(ident, D, (((1,), (1,)), ((), ())),
                        preferred_element_type=jnp.float32)

    t2 = tleaf[...].reshape(BS, LEN)
    ard = jnp.float32(LOG2E * DHA_RD / (R_GAS * TROOM))
    brd = jnp.float32(LOG2E * DHA_RD / R_GAS)
    k25s = (vc25, jm25, tp25)
    for b in range(NB):
        r = 1.0 / t2[b * LEN:(b + 1) * LEN, :]
        e_vj = exp2((-d_vj * LOG2E) * r)
        e_t = exp2((-d_t * LOG2E) * r)
        es = (e_vj, e_vj, e_t)
        for ch in range(3):
            A = T[:, 12 * b + 4 * ch + 0:12 * b + 4 * ch + 1]
            B = T[:, 12 * b + 4 * ch + 1:12 * b + 4 * ch + 2]
            numc = T[:, 12 * b + 4 * ch + 2:12 * b + 4 * ch + 3]
            G = T[:, 12 * b + 4 * ch + 3:12 * b + 4 * ch + 4]
            k25 = k25s[ch][...].reshape(BS, LEN)[b * LEN:(b + 1) * LEN, :]
            res = (k25 * numc * exp2(A * LOG2E - B * r)
                   / (1.0 + G * es[ch]))
            out_ref[ch, pl.ds(b * sub, sub)] = res.reshape(sub)
        rd = rd25[...].reshape(BS, LEN)[b * LEN:(b + 1) * LEN, :]
        out_ref[3, pl.ds(b * sub, sub)] = (
            rd * exp2(ard - brd * r)).reshape(sub)


def kernel(Tleaf, Vcmax25, Jmax25, TPU25, Rd25, dHa_Vcmax, dHa_Jmax, dHa_TPU,
           Topt_Vcmax, Topt_Jmax, Topt_TPU, PIDs, lengths):
    del lengths
    coefs = _sc_gather(PIDs, dHa_Vcmax, dHa_Jmax, dHa_TPU,
                       Topt_Vcmax, Topt_Jmax, Topt_TPU)
    elems = (Tleaf, Vcmax25, Jmax25, TPU25, Rd25)

    chunk = BS * LEN
    eblk = pl.BlockSpec((chunk,), lambda i: (i,))
    cblk = pl.BlockSpec((BS // LEN * CHUNK_C,), lambda i: (i,))
    return pl.pallas_call(
        _tc_body,
        grid=(SEG // BS,),
        in_specs=[eblk] * 5 + [cblk],
        out_specs=pl.BlockSpec((4, chunk), lambda i: (0, i)),
        out_shape=jax.ShapeDtypeStruct((4, TOTAL), jnp.float32),
    )(*elems, coefs)
